# Initial kernel scaffold; baseline (speedup 1.0000x reference)
import jax
import jax.numpy as jnp
from jax.experimental import pallas as pl


def kernel(x, edge_index, W1, b1, W2, b2):
    raise NotImplementedError("write your pallas kernel here")


# speedup vs baseline: 23.7228x; 23.7228x over previous
import jax
import jax.numpy as jnp
from jax import lax
from jax.experimental import pallas as pl
from jax.experimental.pallas import tpu as pltpu
from jax.experimental.pallas import tpu_sc as plsc

N = 10000
E = 320000
F_IN = 128
H = 64

NC = 2
NS = 16
NW = NC * NS

C = 128
EPT = 10240
CH = EPT // C
E_PAD = NW * EPT
N_PAD = 10112
RPT = N_PAD // NS
DW = 8

_MESH = plsc.VectorSubcoreMesh(core_axis_name="c", subcore_axis_name="s")


def _deg_body(dst_hbm, zeros_hbm, ones_hbm, degp_hbm, dst_v, ones_v, deg_sh):
    c = lax.axis_index("c")
    s = lax.axis_index("s")
    w = c * NS + s
    pltpu.sync_copy(dst_hbm.at[w], dst_v)
    pltpu.sync_copy(ones_hbm, ones_v)
    pltpu.sync_copy(zeros_hbm.at[pl.ds(s * RPT, RPT)], deg_sh.at[pl.ds(s * RPT, RPT)])
    plsc.subcore_barrier()

    @pl.loop(0, CH)
    def _chunk(i):
        pltpu.sync_copy(ones_v, deg_sh.at[dst_v.at[i]], add=True)

    plsc.subcore_barrier()
    pltpu.sync_copy(deg_sh.at[pl.ds(s * RPT, RPT)],
                    degp_hbm.at[c, pl.ds(s * RPT, RPT)])


_deg_call = pl.kernel(
    _deg_body,
    out_type=jax.ShapeDtypeStruct((NC, N_PAD, DW), jnp.float32),
    mesh=_MESH,
    scratch_types=[
        pltpu.VMEM((CH, C), jnp.int32),
        pltpu.VMEM((C, DW), jnp.float32),
        pltpu.VMEM_SHARED((N_PAD, DW), jnp.float32),
    ],
    compiler_params=pltpu.CompilerParams(use_tc_tiling_on_sc=False),
)


def _scat_body(g_hbm, src_hbm, dst_hbm, zeros_hbm, accp_hbm,
               src_v, dst_v, rows0, rows1, acc_sh, sem0, sem1):
    c = lax.axis_index("c")
    s = lax.axis_index("s")
    w = c * NS + s
    pltpu.sync_copy(src_hbm.at[w], src_v)
    pltpu.sync_copy(dst_hbm.at[w], dst_v)
    pltpu.sync_copy(zeros_hbm.at[pl.ds(s * RPT, RPT)], acc_sh.at[pl.ds(s * RPT, RPT)])
    plsc.subcore_barrier()

    pltpu.async_copy(g_hbm.at[src_v.at[0]], rows0, sem0)
    pltpu.async_copy(g_hbm.at[src_v.at[1]], rows1, sem1)

    @pl.loop(0, CH // 2)
    def _pair(j):
        i0 = 2 * j
        pltpu.make_async_copy(g_hbm.at[src_v.at[i0]], rows0, sem0).wait()
        pltpu.sync_copy(rows0, acc_sh.at[dst_v.at[i0]], add=True)

        @pl.when(i0 + 2 < CH)
        def _():
            pltpu.async_copy(g_hbm.at[src_v.at[i0 + 2]], rows0, sem0)

        pltpu.make_async_copy(g_hbm.at[src_v.at[i0 + 1]], rows1, sem1).wait()
        pltpu.sync_copy(rows1, acc_sh.at[dst_v.at[i0 + 1]], add=True)

        @pl.when(i0 + 3 < CH)
        def _():
            pltpu.async_copy(g_hbm.at[src_v.at[i0 + 3]], rows1, sem1)

    plsc.subcore_barrier()
    pltpu.sync_copy(acc_sh.at[pl.ds(s * RPT, RPT)],
                    accp_hbm.at[c, pl.ds(s * RPT, RPT)])


_scat_call = pl.kernel(
    _scat_body,
    out_type=jax.ShapeDtypeStruct((NC, N_PAD, H), jnp.float32),
    mesh=_MESH,
    scratch_types=[
        pltpu.VMEM((CH, C), jnp.int32),
        pltpu.VMEM((CH, C), jnp.int32),
        pltpu.VMEM((C, H), jnp.float32),
        pltpu.VMEM((C, H), jnp.float32),
        pltpu.VMEM_SHARED((N_PAD, H), jnp.float32),
        pltpu.SemaphoreType.DMA,
        pltpu.SemaphoreType.DMA,
    ],
    compiler_params=pltpu.CompilerParams(use_tc_tiling_on_sc=False),
)

_R = 400


def _tc1_body(x_ref, w1_ref, degp_ref, g_ref, dinv_ref):
    deg = degp_ref[0, :, 0] + degp_ref[1, :, 0] + 1.0
    dinv = lax.rsqrt(deg)
    h = jnp.dot(x_ref[...], w1_ref[...], preferred_element_type=jnp.float32)
    g_ref[...] = h * dinv[:, None]
    dinv_ref[...] = dinv[:, None]


_tc1_call = pl.pallas_call(
    _tc1_body,
    grid=(N // _R,),
    in_specs=[
        pl.BlockSpec((_R, F_IN), lambda i: (i, 0)),
        pl.BlockSpec((F_IN, H), lambda i: (0, 0)),
        pl.BlockSpec((NC, _R, DW), lambda i: (0, i, 0)),
    ],
    out_specs=[
        pl.BlockSpec((_R, H), lambda i: (i, 0)),
        pl.BlockSpec((_R, 1), lambda i: (i, 0)),
    ],
    out_shape=[
        jax.ShapeDtypeStruct((N, H), jnp.float32),
        jax.ShapeDtypeStruct((N, 1), jnp.float32),
    ],
)


def _tc2_body(accp_ref, g_ref, dinv_ref, b1_ref, w2_ref, b2_ref, out_ref):
    ssum = accp_ref[0] + accp_ref[1] + g_ref[...]
    act = jnp.maximum(dinv_ref[...] * ssum + b1_ref[...], 0.0)
    out_ref[...] = (
        jnp.dot(act, w2_ref[...], preferred_element_type=jnp.float32) + b2_ref[...]
    )


_tc2_call = pl.pallas_call(
    _tc2_body,
    grid=(N // _R,),
    in_specs=[
        pl.BlockSpec((NC, _R, H), lambda i: (0, i, 0)),
        pl.BlockSpec((_R, H), lambda i: (i, 0)),
        pl.BlockSpec((_R, 1), lambda i: (i, 0)),
        pl.BlockSpec((1, H), lambda i: (0, 0)),
        pl.BlockSpec((H, 1), lambda i: (0, 0)),
        pl.BlockSpec((1, 1), lambda i: (0, 0)),
    ],
    out_specs=pl.BlockSpec((_R, 1), lambda i: (i, 0)),
    out_shape=jax.ShapeDtypeStruct((N, 1), jnp.float32),
)


def kernel(x, edge_index, W1, b1, W2, b2):
    pad = E_PAD - E
    src_p = jnp.concatenate(
        [edge_index[0], jnp.zeros((pad,), jnp.int32)]).reshape(NW, CH, C)
    dst_p = jnp.concatenate(
        [edge_index[1], jnp.full((pad,), N, jnp.int32)]).reshape(NW, CH, C)
    zeros_deg = jnp.zeros((N_PAD, DW), jnp.float32)
    ones_c = jnp.ones((C, DW), jnp.float32)
    zeros_acc = jnp.zeros((N_PAD, H), jnp.float32)

    degp = _deg_call(dst_p, zeros_deg, ones_c)
    g, dinv = _tc1_call(x, W1, degp)
    accp = _scat_call(g, src_p, dst_p, zeros_acc)
    return _tc2_call(accp, g, dinv, b1.reshape(1, H), W2, b2.reshape(1, 1))


# speedup vs baseline: 24.5234x; 1.0337x over previous
import jax
import jax.numpy as jnp
from jax import lax
from jax.experimental import pallas as pl
from jax.experimental.pallas import tpu as pltpu
from jax.experimental.pallas import tpu_sc as plsc

N = 10000
E = 320000
F_IN = 128
H = 64

NC = 2
NS = 16
NW = NC * NS

C = 128
EPT = 10240
CH = EPT // C
E_PAD = NW * EPT
N_PAD = 10112
RPT = N_PAD // NS
DW = 8

_MESH = plsc.VectorSubcoreMesh(core_axis_name="c", subcore_axis_name="s")


def _deg_body(dst_hbm, zeros_hbm, ones_hbm, degp_hbm, dst_v, ones_v, deg_sh):
    c = lax.axis_index("c")
    s = lax.axis_index("s")
    w = c * NS + s
    pltpu.sync_copy(dst_hbm.at[w], dst_v)
    pltpu.sync_copy(ones_hbm, ones_v)
    pltpu.sync_copy(zeros_hbm.at[pl.ds(s * RPT, RPT)], deg_sh.at[pl.ds(s * RPT, RPT)])
    plsc.subcore_barrier()

    @pl.loop(0, CH)
    def _chunk(i):
        pltpu.sync_copy(ones_v, deg_sh.at[dst_v.at[i]], add=True)

    plsc.subcore_barrier()
    pltpu.sync_copy(deg_sh.at[pl.ds(s * RPT, RPT)],
                    degp_hbm.at[c, pl.ds(s * RPT, RPT)])


_deg_call = pl.kernel(
    _deg_body,
    out_type=jax.ShapeDtypeStruct((NC, N_PAD, DW), jnp.float32),
    mesh=_MESH,
    scratch_types=[
        pltpu.VMEM((CH, C), jnp.int32),
        pltpu.VMEM((C, DW), jnp.float32),
        pltpu.VMEM_SHARED((N_PAD, DW), jnp.float32),
    ],
    compiler_params=pltpu.CompilerParams(use_tc_tiling_on_sc=False),
)


NBUF = 4


def _scat_body(g_hbm, src_hbm, dst_hbm, zeros_hbm, accp_hbm,
               src_v, dst_v, rows, acc_sh, sems):
    c = lax.axis_index("c")
    s = lax.axis_index("s")
    w = c * NS + s
    pltpu.sync_copy(src_hbm.at[w], src_v)
    pltpu.sync_copy(dst_hbm.at[w], dst_v)
    pltpu.sync_copy(zeros_hbm.at[pl.ds(s * RPT, RPT)], acc_sh.at[pl.ds(s * RPT, RPT)])
    plsc.subcore_barrier()

    for b in range(NBUF):
        pltpu.async_copy(g_hbm.at[src_v.at[b]], rows[b], sems[b])

    @pl.loop(0, CH // NBUF)
    def _round(j):
        i0 = NBUF * j
        for b in range(NBUF):
            i = i0 + b
            pltpu.make_async_copy(g_hbm.at[src_v.at[i]], rows[b], sems[b]).wait()
            pltpu.sync_copy(rows[b], acc_sh.at[dst_v.at[i]], add=True)

            @pl.when(i + NBUF < CH)
            def _():
                pltpu.async_copy(g_hbm.at[src_v.at[i + NBUF]], rows[b], sems[b])

    plsc.subcore_barrier()
    pltpu.sync_copy(acc_sh.at[pl.ds(s * RPT, RPT)],
                    accp_hbm.at[c, pl.ds(s * RPT, RPT)])


_scat_call = pl.kernel(
    _scat_body,
    out_type=jax.ShapeDtypeStruct((NC, N_PAD, H), jnp.float32),
    mesh=_MESH,
    scratch_types=[
        pltpu.VMEM((CH, C), jnp.int32),
        pltpu.VMEM((CH, C), jnp.int32),
        [pltpu.VMEM((C, H), jnp.float32) for _ in range(NBUF)],
        pltpu.VMEM_SHARED((N_PAD, H), jnp.float32),
        [pltpu.SemaphoreType.DMA for _ in range(NBUF)],
    ],
    compiler_params=pltpu.CompilerParams(use_tc_tiling_on_sc=False),
)

_R = 400


def _tch_body(x_ref, w1_ref, h_ref):
    h_ref[...] = jnp.dot(x_ref[...], w1_ref[...],
                         preferred_element_type=jnp.float32)


_tch_call = pl.pallas_call(
    _tch_body,
    grid=(N // _R,),
    in_specs=[
        pl.BlockSpec((_R, F_IN), lambda i: (i, 0)),
        pl.BlockSpec((F_IN, H), lambda i: (0, 0)),
    ],
    out_specs=pl.BlockSpec((_R, H), lambda i: (i, 0)),
    out_shape=jax.ShapeDtypeStruct((N, H), jnp.float32),
)


def _tcg_body(h_ref, degp_ref, g_ref, dinv_ref):
    deg = degp_ref[0, :, 0] + degp_ref[1, :, 0] + 1.0
    dinv = lax.rsqrt(deg)
    g_ref[...] = h_ref[...] * dinv[:, None]
    dinv_ref[...] = dinv[:, None]


_tcg_call = pl.pallas_call(
    _tcg_body,
    grid=(N // _R,),
    in_specs=[
        pl.BlockSpec((_R, H), lambda i: (i, 0)),
        pl.BlockSpec((NC, _R, DW), lambda i: (0, i, 0)),
    ],
    out_specs=[
        pl.BlockSpec((_R, H), lambda i: (i, 0)),
        pl.BlockSpec((_R, 1), lambda i: (i, 0)),
    ],
    out_shape=[
        jax.ShapeDtypeStruct((N, H), jnp.float32),
        jax.ShapeDtypeStruct((N, 1), jnp.float32),
    ],
)


def _tc2_body(accp_ref, g_ref, dinv_ref, b1_ref, w2_ref, b2_ref, out_ref):
    ssum = accp_ref[0] + accp_ref[1] + g_ref[...]
    act = jnp.maximum(dinv_ref[...] * ssum + b1_ref[...], 0.0)
    out_ref[...] = (
        jnp.dot(act, w2_ref[...], preferred_element_type=jnp.float32) + b2_ref[...]
    )


_tc2_call = pl.pallas_call(
    _tc2_body,
    grid=(N // _R,),
    in_specs=[
        pl.BlockSpec((NC, _R, H), lambda i: (0, i, 0)),
        pl.BlockSpec((_R, H), lambda i: (i, 0)),
        pl.BlockSpec((_R, 1), lambda i: (i, 0)),
        pl.BlockSpec((1, H), lambda i: (0, 0)),
        pl.BlockSpec((H, 1), lambda i: (0, 0)),
        pl.BlockSpec((1, 1), lambda i: (0, 0)),
    ],
    out_specs=pl.BlockSpec((_R, 1), lambda i: (i, 0)),
    out_shape=jax.ShapeDtypeStruct((N, 1), jnp.float32),
)


def kernel(x, edge_index, W1, b1, W2, b2):
    pad = E_PAD - E
    src_p = jnp.concatenate(
        [edge_index[0], jnp.zeros((pad,), jnp.int32)]).reshape(NW, CH, C)
    dst_p = jnp.concatenate(
        [edge_index[1], jnp.full((pad,), N, jnp.int32)]).reshape(NW, CH, C)
    zeros_deg = jnp.zeros((N_PAD, DW), jnp.float32)
    ones_c = jnp.ones((C, DW), jnp.float32)
    zeros_acc = jnp.zeros((N_PAD, H), jnp.float32)

    h = _tch_call(x, W1)
    degp = _deg_call(dst_p, zeros_deg, ones_c)
    g, dinv = _tcg_call(h, degp)
    accp = _scat_call(g, src_p, dst_p, zeros_acc)
    return _tc2_call(accp, g, dinv, b1.reshape(1, H), W2, b2.reshape(1, 1))


# speedup vs baseline: 26.6568x; 1.0870x over previous
import jax
import jax.numpy as jnp
from jax import lax
from jax.experimental import pallas as pl
from jax.experimental.pallas import tpu as pltpu
from jax.experimental.pallas import tpu_sc as plsc

N = 10000
E = 320000
F_IN = 128
H = 64

NC = 2
NS = 16
NW = NC * NS

C = 128
EPT = 10240
CH = EPT // C
E_PAD = NW * EPT
N_PAD = 10112
RPT = N_PAD // NS
DW = 8

_MESH = plsc.VectorSubcoreMesh(core_axis_name="c", subcore_axis_name="s")


def _deg_body(dst_hbm, zeros_hbm, ones_hbm, degp_hbm, dst_v, ones_v, deg_sh):
    c = lax.axis_index("c")
    s = lax.axis_index("s")
    w = c * NS + s
    pltpu.sync_copy(dst_hbm.at[w], dst_v)
    pltpu.sync_copy(ones_hbm, ones_v)
    pltpu.sync_copy(zeros_hbm.at[pl.ds(s * RPT, RPT)], deg_sh.at[pl.ds(s * RPT, RPT)])
    plsc.subcore_barrier()

    @pl.loop(0, CH)
    def _chunk(i):
        pltpu.sync_copy(ones_v, deg_sh.at[dst_v.at[i]], add=True)

    plsc.subcore_barrier()
    pltpu.sync_copy(deg_sh.at[pl.ds(s * RPT, RPT)],
                    degp_hbm.at[c, pl.ds(s * RPT, RPT)])


_deg_call = pl.kernel(
    _deg_body,
    out_type=jax.ShapeDtypeStruct((NC, N_PAD, DW), jnp.float32),
    mesh=_MESH,
    scratch_types=[
        pltpu.VMEM((CH, C), jnp.int32),
        pltpu.VMEM((C, DW), jnp.float32),
        pltpu.VMEM_SHARED((N_PAD, DW), jnp.float32),
    ],
    compiler_params=pltpu.CompilerParams(use_tc_tiling_on_sc=False),
)


NBUF = 4


def _scat_body(g_hbm, src_hbm, dst_hbm, zeros_hbm, accp_hbm,
               src_v, dst_v, rows, acc_sh, sems):
    c = lax.axis_index("c")
    s = lax.axis_index("s")
    w = c * NS + s
    pltpu.sync_copy(src_hbm.at[w], src_v)
    pltpu.sync_copy(dst_hbm.at[w], dst_v)
    pltpu.sync_copy(zeros_hbm.at[pl.ds(s * RPT, RPT)], acc_sh.at[pl.ds(s * RPT, RPT)])
    plsc.subcore_barrier()

    for b in range(NBUF):
        pltpu.async_copy(g_hbm.at[src_v.at[b]], rows[b], sems[b])

    @pl.loop(0, CH // NBUF)
    def _round(j):
        i0 = NBUF * j
        for b in range(NBUF):
            i = i0 + b
            pltpu.make_async_copy(g_hbm.at[src_v.at[i]], rows[b], sems[b]).wait()
            pltpu.sync_copy(rows[b], acc_sh.at[dst_v.at[i]], add=True)

            @pl.when(i + NBUF < CH)
            def _():
                pltpu.async_copy(g_hbm.at[src_v.at[i + NBUF]], rows[b], sems[b])

    plsc.subcore_barrier()
    pltpu.sync_copy(acc_sh.at[pl.ds(s * RPT, RPT)],
                    accp_hbm.at[c, pl.ds(s * RPT, RPT)])


_scat_call = pl.kernel(
    _scat_body,
    out_type=jax.ShapeDtypeStruct((NC, N_PAD, H), jnp.float32),
    mesh=_MESH,
    scratch_types=[
        pltpu.VMEM((CH, C), jnp.int32),
        pltpu.VMEM((CH, C), jnp.int32),
        [pltpu.VMEM((C, H), jnp.float32) for _ in range(NBUF)],
        pltpu.VMEM_SHARED((N_PAD, H), jnp.float32),
        [pltpu.SemaphoreType.DMA for _ in range(NBUF)],
    ],
    compiler_params=pltpu.CompilerParams(use_tc_tiling_on_sc=False),
)

_R = 400


def _tch_body(x_ref, w1_ref, h_ref):
    h_ref[...] = jnp.dot(x_ref[...], w1_ref[...],
                         preferred_element_type=jnp.float32)


_tch_call = pl.pallas_call(
    _tch_body,
    out_shape=jax.ShapeDtypeStruct((N, H), jnp.float32),
)


def _tcg_body(h_ref, degp_ref, g_ref, dinv_ref):
    deg = degp_ref[0, 0:N, 0] + degp_ref[1, 0:N, 0] + 1.0
    dinv = lax.rsqrt(deg)
    g_ref[...] = h_ref[...] * dinv[:, None]
    dinv_ref[...] = dinv[:, None]


_tcg_call = pl.pallas_call(
    _tcg_body,
    out_shape=[
        jax.ShapeDtypeStruct((N, H), jnp.float32),
        jax.ShapeDtypeStruct((N, 1), jnp.float32),
    ],
)


def _tc2_body(accp_ref, g_ref, dinv_ref, b1_ref, w2_ref, b2_ref, out_ref):
    ssum = accp_ref[0, 0:N, :] + accp_ref[1, 0:N, :] + g_ref[...]
    act = jnp.maximum(dinv_ref[...] * ssum + b1_ref[...], 0.0)
    out_ref[...] = (
        jnp.dot(act, w2_ref[...], preferred_element_type=jnp.float32) + b2_ref[...]
    )


_tc2_call = pl.pallas_call(
    _tc2_body,
    out_shape=jax.ShapeDtypeStruct((N, 1), jnp.float32),
)


def kernel(x, edge_index, W1, b1, W2, b2):
    pad = E_PAD - E
    pad_dst = N + (jnp.arange(pad, dtype=jnp.int32) % (N_PAD - N))
    src_p = jnp.concatenate(
        [edge_index[0], jnp.zeros((pad,), jnp.int32)]).reshape(NW, CH, C)
    dst_p = jnp.concatenate(
        [edge_index[1], pad_dst]).reshape(NW, CH, C)
    zeros_deg = jnp.zeros((N_PAD, DW), jnp.float32)
    ones_c = jnp.ones((C, DW), jnp.float32)
    zeros_acc = jnp.zeros((N_PAD, H), jnp.float32)

    h = _tch_call(x, W1)
    degp = _deg_call(dst_p, zeros_deg, ones_c)
    g, dinv = _tcg_call(h, degp)
    accp = _scat_call(g, src_p, dst_p, zeros_acc)
    return _tc2_call(accp, g, dinv, b1.reshape(1, H), W2, b2.reshape(1, 1))


# speedup vs baseline: 44.4036x; 1.6658x over previous
import jax
import jax.numpy as jnp
from jax import lax
from jax.experimental import pallas as pl
from jax.experimental.pallas import tpu as pltpu
from jax.experimental.pallas import tpu_sc as plsc

N = 10000
E = 320000
F_IN = 128
H = 64

NC = 2
NS = 16
NW = NC * NS

C = 128
EPT = 10240
CH = EPT // C
E_PAD = NW * EPT
N_PAD = 10112
RPT = N_PAD // NS
DW = 8

_MESH = plsc.VectorSubcoreMesh(core_axis_name="c", subcore_axis_name="s")


def _deg_body(dst_hbm, zeros_hbm, ones_hbm, degp_hbm, dst_v, ones_v, deg_sh):
    c = lax.axis_index("c")
    s = lax.axis_index("s")
    w = c * NS + s
    pltpu.sync_copy(dst_hbm.at[w], dst_v)
    pltpu.sync_copy(ones_hbm, ones_v)
    pltpu.sync_copy(zeros_hbm.at[pl.ds(s * RPT, RPT)], deg_sh.at[pl.ds(s * RPT, RPT)])
    plsc.subcore_barrier()

    @pl.loop(0, CH)
    def _chunk(i):
        pltpu.sync_copy(ones_v, deg_sh.at[dst_v.at[i]], add=True)

    plsc.subcore_barrier()
    pltpu.sync_copy(deg_sh.at[pl.ds(s * RPT, RPT)],
                    degp_hbm.at[c, pl.ds(s * RPT, RPT)])


_deg_call = pl.kernel(
    _deg_body,
    out_type=jax.ShapeDtypeStruct((NC, N_PAD, DW), jnp.float32),
    mesh=_MESH,
    scratch_types=[
        pltpu.VMEM((CH, C), jnp.int32),
        pltpu.VMEM((C, DW), jnp.float32),
        pltpu.VMEM_SHARED((N_PAD, DW), jnp.float32),
    ],
    compiler_params=pltpu.CompilerParams(use_tc_tiling_on_sc=False),
)


NBUF = 2


GPT = N // NS


ZR = RPT // 4


def _scat_body(g_hbm, src_hbm, dst_hbm, accp_hbm,
               src_v, dst_v, rows, zbuf, g_sh, acc_sh, sems):
    c = lax.axis_index("c")
    s = lax.axis_index("s")
    w = c * NS + s
    pltpu.sync_copy(src_hbm.at[w], src_v)
    pltpu.sync_copy(dst_hbm.at[w], dst_v)
    pltpu.sync_copy(g_hbm.at[pl.ds(s * GPT, GPT)], g_sh.at[pl.ds(s * GPT, GPT)])

    z16 = jnp.zeros((16,), jnp.float32)

    @pl.loop(0, ZR)
    def _zfill(r):
        for q in range(H // 16):
            zbuf[r, pl.ds(16 * q, 16)] = z16

    for k in range(4):
        pltpu.sync_copy(zbuf, acc_sh.at[pl.ds(s * RPT + k * ZR, ZR)])
    plsc.subcore_barrier()

    for b in range(NBUF):
        pltpu.async_copy(g_sh.at[src_v.at[b]], rows[b], sems[b])

    @pl.loop(0, CH // NBUF)
    def _round(j):
        i0 = NBUF * j
        for b in range(NBUF):
            i = i0 + b
            pltpu.make_async_copy(g_sh.at[src_v.at[i]], rows[b], sems[b]).wait()
            pltpu.sync_copy(rows[b], acc_sh.at[dst_v.at[i]], add=True)

            @pl.when(i + NBUF < CH)
            def _():
                pltpu.async_copy(g_sh.at[src_v.at[i + NBUF]], rows[b], sems[b])

    plsc.subcore_barrier()
    pltpu.sync_copy(acc_sh.at[pl.ds(s * RPT, RPT)],
                    accp_hbm.at[c, pl.ds(s * RPT, RPT)])


_scat_call = pl.kernel(
    _scat_body,
    out_type=jax.ShapeDtypeStruct((NC, N_PAD, H), jnp.float32),
    mesh=_MESH,
    scratch_types=[
        pltpu.VMEM((CH, C), jnp.int32),
        pltpu.VMEM((CH, C), jnp.int32),
        [pltpu.VMEM((C, H), jnp.float32) for _ in range(NBUF)],
        pltpu.VMEM((ZR, H), jnp.float32),
        pltpu.VMEM_SHARED((N, H), jnp.float32),
        pltpu.VMEM_SHARED((N_PAD, H), jnp.float32),
        [pltpu.SemaphoreType.DMA for _ in range(NBUF)],
    ],
    compiler_params=pltpu.CompilerParams(use_tc_tiling_on_sc=False),
)

_R = 400


def _tch_body(x_ref, w1_ref, h_ref):
    h_ref[...] = jnp.dot(x_ref[...], w1_ref[...],
                         preferred_element_type=jnp.float32)


_tch_call = pl.pallas_call(
    _tch_body,
    out_shape=jax.ShapeDtypeStruct((N, H), jnp.float32),
)


def _tcg_body(h_ref, degp_ref, g_ref, dinv_ref):
    deg = degp_ref[0, 0:N, 0] + degp_ref[1, 0:N, 0] + 1.0
    dinv = lax.rsqrt(deg)
    g_ref[...] = h_ref[...] * dinv[:, None]
    dinv_ref[...] = dinv[:, None]


_tcg_call = pl.pallas_call(
    _tcg_body,
    out_shape=[
        jax.ShapeDtypeStruct((N, H), jnp.float32),
        jax.ShapeDtypeStruct((N, 1), jnp.float32),
    ],
)


def _tc2_body(accp_ref, g_ref, dinv_ref, b1_ref, w2_ref, b2_ref, out_ref):
    ssum = accp_ref[0, 0:N, :] + accp_ref[1, 0:N, :] + g_ref[...]
    act = jnp.maximum(dinv_ref[...] * ssum + b1_ref[...], 0.0)
    out_ref[...] = (
        jnp.dot(act, w2_ref[...], preferred_element_type=jnp.float32) + b2_ref[...]
    )


_tc2_call = pl.pallas_call(
    _tc2_body,
    out_shape=jax.ShapeDtypeStruct((N, 1), jnp.float32),
)


def kernel(x, edge_index, W1, b1, W2, b2):
    pad = E_PAD - E
    pad_dst = N + (jnp.arange(pad, dtype=jnp.int32) % (N_PAD - N))
    src_p = jnp.concatenate(
        [edge_index[0], jnp.zeros((pad,), jnp.int32)]).reshape(NW, CH, C)
    dst_p = jnp.concatenate(
        [edge_index[1], pad_dst]).reshape(NW, CH, C)
    zeros_deg = jnp.zeros((N_PAD, DW), jnp.float32)
    ones_c = jnp.ones((C, DW), jnp.float32)

    h = _tch_call(x, W1)
    degp = _deg_call(dst_p, zeros_deg, ones_c)
    g, dinv = _tcg_call(h, degp)
    accp = _scat_call(g, src_p, dst_p)
    return _tc2_call(accp, g, dinv, b1.reshape(1, H), W2, b2.reshape(1, 1))


# speedup vs baseline: 47.7337x; 1.0750x over previous
import jax
import jax.numpy as jnp
from jax import lax
from jax.experimental import pallas as pl
from jax.experimental.pallas import tpu as pltpu
from jax.experimental.pallas import tpu_sc as plsc

N = 10000
E = 320000
F_IN = 128
H = 64

NC = 2
NS = 16
NW = NC * NS

C = 128
NCH = E // C
CPW = NCH // NW
XTRA = NCH - CPW * NW
MCH = CPW + 1

N_PAD = 10112
RPT = N_PAD // NS
GPT = N // NS
ZR = RPT // 4
DW = 8

_MESH = plsc.VectorSubcoreMesh(core_axis_name="c", subcore_axis_name="s")


def _stage_idx(e3_hbm, row, idx_v, base, extra):
    pltpu.sync_copy(e3_hbm.at[row, pl.ds(base, CPW)], idx_v.at[pl.ds(0, CPW)])

    @pl.when(extra)
    def _():
        pltpu.sync_copy(e3_hbm.at[row, pl.ds(base + CPW, 1)],
                        idx_v.at[pl.ds(CPW, 1)])


def _deg_body(e3_hbm, zeros_hbm, ones_hbm, degp_hbm, dst_v, ones_v, deg_sh):
    c = lax.axis_index("c")
    s = lax.axis_index("s")
    w = c * NS + s
    base = w * CPW + jnp.minimum(w, XTRA)
    extra = w < XTRA
    _stage_idx(e3_hbm, 1, dst_v, base, extra)
    pltpu.sync_copy(ones_hbm, ones_v)
    pltpu.sync_copy(zeros_hbm.at[pl.ds(s * RPT, RPT)], deg_sh.at[pl.ds(s * RPT, RPT)])
    plsc.subcore_barrier()

    @pl.loop(0, CPW)
    def _chunk(i):
        pltpu.sync_copy(ones_v, deg_sh.at[dst_v.at[i]], add=True)

    @pl.when(extra)
    def _():
        pltpu.sync_copy(ones_v, deg_sh.at[dst_v.at[CPW]], add=True)

    plsc.subcore_barrier()
    pltpu.sync_copy(deg_sh.at[pl.ds(s * RPT, RPT)],
                    degp_hbm.at[c, pl.ds(s * RPT, RPT)])


_deg_call = pl.kernel(
    _deg_body,
    out_type=jax.ShapeDtypeStruct((NC, N_PAD, DW), jnp.float32),
    mesh=_MESH,
    scratch_types=[
        pltpu.VMEM((MCH, C), jnp.int32),
        pltpu.VMEM((C, DW), jnp.float32),
        pltpu.VMEM_SHARED((N_PAD, DW), jnp.float32),
    ],
    compiler_params=pltpu.CompilerParams(use_tc_tiling_on_sc=False),
)


def _scat_body(g_hbm, e3_hbm, accp_hbm,
               src_v, dst_v, rows, zbuf, g_sh, acc_sh, sems):
    c = lax.axis_index("c")
    s = lax.axis_index("s")
    w = c * NS + s
    base = w * CPW + jnp.minimum(w, XTRA)
    extra = w < XTRA
    count = CPW + extra.astype(jnp.int32)
    _stage_idx(e3_hbm, 0, src_v, base, extra)
    _stage_idx(e3_hbm, 1, dst_v, base, extra)
    pltpu.sync_copy(g_hbm.at[pl.ds(s * GPT, GPT)], g_sh.at[pl.ds(s * GPT, GPT)])

    z16 = jnp.zeros((16,), jnp.float32)

    @pl.loop(0, ZR)
    def _zfill(r):
        for q in range(H // 16):
            zbuf[r, pl.ds(16 * q, 16)] = z16

    for k in range(4):
        pltpu.sync_copy(zbuf, acc_sh.at[pl.ds(s * RPT + k * ZR, ZR)])
    plsc.subcore_barrier()

    for b in range(2):
        pltpu.async_copy(g_sh.at[src_v.at[b]], rows[b], sems[b])

    @pl.loop(0, CPW // 2)
    def _pair(j):
        i0 = 2 * j
        for b in range(2):
            i = i0 + b
            pltpu.make_async_copy(g_sh.at[src_v.at[i]], rows[b], sems[b]).wait()
            pltpu.sync_copy(rows[b], acc_sh.at[dst_v.at[i]], add=True)

            @pl.when(i + 2 < count)
            def _():
                pltpu.async_copy(g_sh.at[src_v.at[i + 2]], rows[b], sems[b])

    @pl.when(extra)
    def _():
        pltpu.make_async_copy(g_sh.at[src_v.at[CPW]], rows[0], sems[0]).wait()
        pltpu.sync_copy(rows[0], acc_sh.at[dst_v.at[CPW]], add=True)

    plsc.subcore_barrier()
    pltpu.sync_copy(acc_sh.at[pl.ds(s * RPT, RPT)],
                    accp_hbm.at[c, pl.ds(s * RPT, RPT)])


_scat_call = pl.kernel(
    _scat_body,
    out_type=jax.ShapeDtypeStruct((NC, N_PAD, H), jnp.float32),
    mesh=_MESH,
    scratch_types=[
        pltpu.VMEM((MCH, C), jnp.int32),
        pltpu.VMEM((MCH, C), jnp.int32),
        [pltpu.VMEM((C, H), jnp.float32) for _ in range(2)],
        pltpu.VMEM((ZR, H), jnp.float32),
        pltpu.VMEM_SHARED((N, H), jnp.float32),
        pltpu.VMEM_SHARED((N_PAD, H), jnp.float32),
        [pltpu.SemaphoreType.DMA for _ in range(2)],
    ],
    compiler_params=pltpu.CompilerParams(use_tc_tiling_on_sc=False),
)


def _tch_body(x_ref, w1_ref, h_ref):
    h_ref[...] = jnp.dot(x_ref[...], w1_ref[...],
                         preferred_element_type=jnp.float32)


_tch_call = pl.pallas_call(
    _tch_body,
    out_shape=jax.ShapeDtypeStruct((N, H), jnp.float32),
)


def _tcg_body(h_ref, degp_ref, g_ref, dinv_ref):
    deg = degp_ref[0, 0:N, 0] + degp_ref[1, 0:N, 0] + 1.0
    dinv = lax.rsqrt(deg)
    g_ref[...] = h_ref[...] * dinv[:, None]
    dinv_ref[...] = dinv[:, None]


_tcg_call = pl.pallas_call(
    _tcg_body,
    out_shape=[
        jax.ShapeDtypeStruct((N, H), jnp.float32),
        jax.ShapeDtypeStruct((N, 1), jnp.float32),
    ],
)


def _tc2_body(accp_ref, g_ref, dinv_ref, b1_ref, w2_ref, b2_ref, out_ref):
    ssum = accp_ref[0, 0:N, :] + accp_ref[1, 0:N, :] + g_ref[...]
    act = jnp.maximum(dinv_ref[...] * ssum + b1_ref[...], 0.0)
    out_ref[...] = (
        jnp.dot(act, w2_ref[...], preferred_element_type=jnp.float32) + b2_ref[...]
    )


_tc2_call = pl.pallas_call(
    _tc2_body,
    out_shape=jax.ShapeDtypeStruct((N, 1), jnp.float32),
)


def kernel(x, edge_index, W1, b1, W2, b2):
    e3 = edge_index.reshape(2, NCH, C)
    zeros_deg = jnp.zeros((N_PAD, DW), jnp.float32)
    ones_c = jnp.ones((C, DW), jnp.float32)

    h = _tch_call(x, W1)
    degp = _deg_call(e3, zeros_deg, ones_c)
    g, dinv = _tcg_call(h, degp)
    accp = _scat_call(g, e3)
    return _tc2_call(accp, g, dinv, b1.reshape(1, H), W2, b2.reshape(1, 1))


# speedup vs baseline: 47.8195x; 1.0018x over previous
import jax
import jax.numpy as jnp
from jax import lax
from jax.experimental import pallas as pl
from jax.experimental.pallas import tpu as pltpu
from jax.experimental.pallas import tpu_sc as plsc

N = 10000
E = 320000
F_IN = 128
H = 64

NC = 2
NS = 16
NW = NC * NS

C = 128
NCH = E // C
CPW = NCH // NW
XTRA = NCH - CPW * NW
MCH = CPW + 1

N_PAD = 10112
RPT = N_PAD // NS
GPT = N // NS
ZR = RPT // 4
DW = 16

_MESH = plsc.VectorSubcoreMesh(core_axis_name="c", subcore_axis_name="s")


def _stage_idx(e3_hbm, row, idx_v, base, extra):
    pltpu.sync_copy(e3_hbm.at[row, pl.ds(base, CPW)], idx_v.at[pl.ds(0, CPW)])

    @pl.when(extra)
    def _():
        pltpu.sync_copy(e3_hbm.at[row, pl.ds(base + CPW, 1)],
                        idx_v.at[pl.ds(CPW, 1)])


def _deg_body(e3_hbm, zeros_hbm, ones_hbm, degp_hbm, dst_v, ones_v, deg_sh):
    c = lax.axis_index("c")
    s = lax.axis_index("s")
    w = c * NS + s
    base = w * CPW + jnp.minimum(w, XTRA)
    extra = w < XTRA
    _stage_idx(e3_hbm, 1, dst_v, base, extra)
    pltpu.sync_copy(ones_hbm, ones_v)
    pltpu.sync_copy(zeros_hbm.at[pl.ds(s * RPT, RPT)], deg_sh.at[pl.ds(s * RPT, RPT)])
    plsc.subcore_barrier()

    @pl.loop(0, CPW)
    def _chunk(i):
        pltpu.sync_copy(ones_v, deg_sh.at[dst_v.at[i]], add=True)

    @pl.when(extra)
    def _():
        pltpu.sync_copy(ones_v, deg_sh.at[dst_v.at[CPW]], add=True)

    plsc.subcore_barrier()
    pltpu.sync_copy(deg_sh.at[pl.ds(s * RPT, RPT)],
                    degp_hbm.at[c, pl.ds(s * RPT, RPT)])


_deg_call = pl.kernel(
    _deg_body,
    out_type=jax.ShapeDtypeStruct((NC, N_PAD, DW), jnp.int16),
    mesh=_MESH,
    scratch_types=[
        pltpu.VMEM((MCH, C), jnp.int32),
        pltpu.VMEM((C, DW), jnp.int16),
        pltpu.VMEM_SHARED((N_PAD, DW), jnp.int16),
    ],
    compiler_params=pltpu.CompilerParams(use_tc_tiling_on_sc=False),
)


def _scat_body(g_hbm, e3_hbm, accp_hbm,
               src_v, dst_v, rows, zbuf, g_sh, acc_sh, sems):
    c = lax.axis_index("c")
    s = lax.axis_index("s")
    w = c * NS + s
    base = w * CPW + jnp.minimum(w, XTRA)
    extra = w < XTRA
    count = CPW + extra.astype(jnp.int32)
    _stage_idx(e3_hbm, 0, src_v, base, extra)
    _stage_idx(e3_hbm, 1, dst_v, base, extra)
    pltpu.sync_copy(g_hbm.at[pl.ds(s * GPT, GPT)], g_sh.at[pl.ds(s * GPT, GPT)])

    z16 = jnp.zeros((16,), jnp.float32)

    @pl.loop(0, ZR)
    def _zfill(r):
        for q in range(H // 16):
            zbuf[r, pl.ds(16 * q, 16)] = z16

    for k in range(4):
        pltpu.sync_copy(zbuf, acc_sh.at[pl.ds(s * RPT + k * ZR, ZR)])
    plsc.subcore_barrier()

    for b in range(2):
        pltpu.async_copy(g_sh.at[src_v.at[b]], rows[b], sems[b])

    @pl.loop(0, CPW // 2)
    def _pair(j):
        i0 = 2 * j
        for b in range(2):
            i = i0 + b
            pltpu.make_async_copy(g_sh.at[src_v.at[i]], rows[b], sems[b]).wait()
            pltpu.sync_copy(rows[b], acc_sh.at[dst_v.at[i]], add=True)

            @pl.when(i + 2 < count)
            def _():
                pltpu.async_copy(g_sh.at[src_v.at[i + 2]], rows[b], sems[b])

    @pl.when(extra)
    def _():
        pltpu.make_async_copy(g_sh.at[src_v.at[CPW]], rows[0], sems[0]).wait()
        pltpu.sync_copy(rows[0], acc_sh.at[dst_v.at[CPW]], add=True)

    plsc.subcore_barrier()
    pltpu.sync_copy(acc_sh.at[pl.ds(s * RPT, RPT)],
                    accp_hbm.at[c, pl.ds(s * RPT, RPT)])


_scat_call = pl.kernel(
    _scat_body,
    out_type=jax.ShapeDtypeStruct((NC, N_PAD, H), jnp.float32),
    mesh=_MESH,
    scratch_types=[
        pltpu.VMEM((MCH, C), jnp.int32),
        pltpu.VMEM((MCH, C), jnp.int32),
        [pltpu.VMEM((C, H), jnp.float32) for _ in range(2)],
        pltpu.VMEM((ZR, H), jnp.float32),
        pltpu.VMEM_SHARED((N, H), jnp.float32),
        pltpu.VMEM_SHARED((N_PAD, H), jnp.float32),
        [pltpu.SemaphoreType.DMA for _ in range(2)],
    ],
    compiler_params=pltpu.CompilerParams(use_tc_tiling_on_sc=False),
)


def _tch_body(x_ref, w1_ref, h_ref):
    h_ref[...] = jnp.dot(x_ref[...], w1_ref[...],
                         preferred_element_type=jnp.float32)


_tch_call = pl.pallas_call(
    _tch_body,
    out_shape=jax.ShapeDtypeStruct((N, H), jnp.float32),
)


def _tcg_body(h_ref, degp_ref, g_ref, dinv_ref):
    deg = (degp_ref[0, 0:N, 0] + degp_ref[1, 0:N, 0]).astype(jnp.float32) + 1.0
    dinv = lax.rsqrt(deg)
    g_ref[...] = h_ref[...] * dinv[:, None]
    dinv_ref[...] = dinv[:, None]


_tcg_call = pl.pallas_call(
    _tcg_body,
    out_shape=[
        jax.ShapeDtypeStruct((N, H), jnp.float32),
        jax.ShapeDtypeStruct((N, 1), jnp.float32),
    ],
)


def _tc2_body(accp_ref, g_ref, dinv_ref, b1_ref, w2_ref, b2_ref, out_ref):
    ssum = accp_ref[0, 0:N, :] + accp_ref[1, 0:N, :] + g_ref[...]
    act = jnp.maximum(dinv_ref[...] * ssum + b1_ref[...], 0.0)
    out_ref[...] = (
        jnp.dot(act, w2_ref[...], preferred_element_type=jnp.float32) + b2_ref[...]
    )


_tc2_call = pl.pallas_call(
    _tc2_body,
    out_shape=jax.ShapeDtypeStruct((N, 1), jnp.float32),
)


def kernel(x, edge_index, W1, b1, W2, b2):
    e3 = edge_index.reshape(2, NCH, C)
    zeros_deg = jnp.zeros((N_PAD, DW), jnp.int16)
    ones_c = jnp.ones((C, DW), jnp.int16)

    h = _tch_call(x, W1)
    degp = _deg_call(e3, zeros_deg, ones_c)
    g, dinv = _tcg_call(h, degp)
    accp = _scat_call(g, e3)
    return _tc2_call(accp, g, dinv, b1.reshape(1, H), W2, b2.reshape(1, 1))


# speedup vs baseline: 48.5193x; 1.0146x over previous
import jax
import jax.numpy as jnp
from jax import lax
from jax.experimental import pallas as pl
from jax.experimental.pallas import tpu as pltpu
from jax.experimental.pallas import tpu_sc as plsc

N = 10000
E = 320000
F_IN = 128
H = 64

NC = 2
NS = 16
NW = NC * NS

C = 128
NCH = E // C
CPW = NCH // NW
XTRA = NCH - CPW * NW
MCH = CPW + 1

N_PAD = 10112
RPT = N_PAD // NS
GPT = N // NS
ZR = RPT // 4
DW = 16

_MESH = plsc.VectorSubcoreMesh(core_axis_name="c", subcore_axis_name="s")


def _stage_idx(e3_hbm, row, idx_v, base, extra):
    pltpu.sync_copy(e3_hbm.at[row, pl.ds(base, CPW)], idx_v.at[pl.ds(0, CPW)])

    @pl.when(extra)
    def _():
        pltpu.sync_copy(e3_hbm.at[row, pl.ds(base + CPW, 1)],
                        idx_v.at[pl.ds(CPW, 1)])


def _deg_body(e3_hbm, zeros_hbm, ones_hbm, degp_hbm, dst_v, ones_v, deg_sh):
    c = lax.axis_index("c")
    s = lax.axis_index("s")
    w = c * NS + s
    base = w * CPW + jnp.minimum(w, XTRA)
    extra = w < XTRA
    _stage_idx(e3_hbm, 1, dst_v, base, extra)
    pltpu.sync_copy(ones_hbm, ones_v)
    pltpu.sync_copy(zeros_hbm.at[pl.ds(s * RPT, RPT)], deg_sh.at[pl.ds(s * RPT, RPT)])
    plsc.subcore_barrier()

    @pl.loop(0, CPW)
    def _chunk(i):
        pltpu.sync_copy(ones_v, deg_sh.at[dst_v.at[i]], add=True)

    @pl.when(extra)
    def _():
        pltpu.sync_copy(ones_v, deg_sh.at[dst_v.at[CPW]], add=True)

    plsc.subcore_barrier()
    pltpu.sync_copy(deg_sh.at[pl.ds(s * RPT, RPT)],
                    degp_hbm.at[c, pl.ds(s * RPT, RPT)])


_deg_call = pl.kernel(
    _deg_body,
    out_type=jax.ShapeDtypeStruct((NC, N_PAD, DW), jnp.int16),
    mesh=_MESH,
    scratch_types=[
        pltpu.VMEM((MCH, C), jnp.int32),
        pltpu.VMEM((C, DW), jnp.int16),
        pltpu.VMEM_SHARED((N_PAD, DW), jnp.int16),
    ],
    compiler_params=pltpu.CompilerParams(use_tc_tiling_on_sc=False),
)


def _scat_body(g_hbm, e3_hbm, accp_hbm,
               src_v, dst_v, rows, zbuf, g_sh, acc_sh, sems):
    c = lax.axis_index("c")
    s = lax.axis_index("s")
    w = c * NS + s
    base = w * CPW + jnp.minimum(w, XTRA)
    extra = w < XTRA
    count = CPW + extra.astype(jnp.int32)
    _stage_idx(e3_hbm, 0, src_v, base, extra)
    _stage_idx(e3_hbm, 1, dst_v, base, extra)
    gstage = pltpu.async_copy(g_hbm.at[pl.ds(s * GPT, GPT)],
                              g_sh.at[pl.ds(s * GPT, GPT)], sems[0])

    z16 = jnp.zeros((16,), jnp.float32)

    @pl.loop(0, ZR)
    def _zfill(r):
        for q in range(H // 16):
            zbuf[r, pl.ds(16 * q, 16)] = z16

    for k in range(4):
        pltpu.sync_copy(zbuf, acc_sh.at[pl.ds(s * RPT + k * ZR, ZR)])
    gstage.wait()
    plsc.subcore_barrier()

    for b in range(2):
        pltpu.async_copy(g_sh.at[src_v.at[b]], rows[b], sems[b])

    @pl.loop(0, CPW // 2)
    def _pair(j):
        i0 = 2 * j
        for b in range(2):
            i = i0 + b
            pltpu.make_async_copy(g_sh.at[src_v.at[i]], rows[b], sems[b]).wait()
            pltpu.sync_copy(rows[b], acc_sh.at[dst_v.at[i]], add=True)

            @pl.when(i + 2 < count)
            def _():
                pltpu.async_copy(g_sh.at[src_v.at[i + 2]], rows[b], sems[b])

    @pl.when(extra)
    def _():
        pltpu.make_async_copy(g_sh.at[src_v.at[CPW]], rows[0], sems[0]).wait()
        pltpu.sync_copy(rows[0], acc_sh.at[dst_v.at[CPW]], add=True)

    plsc.subcore_barrier()
    pltpu.sync_copy(acc_sh.at[pl.ds(s * RPT, RPT)],
                    accp_hbm.at[c, pl.ds(s * RPT, RPT)])


_scat_call = pl.kernel(
    _scat_body,
    out_type=jax.ShapeDtypeStruct((NC, N_PAD, H), jnp.float32),
    mesh=_MESH,
    scratch_types=[
        pltpu.VMEM((MCH, C), jnp.int32),
        pltpu.VMEM((MCH, C), jnp.int32),
        [pltpu.VMEM((C, H), jnp.float32) for _ in range(2)],
        pltpu.VMEM((ZR, H), jnp.float32),
        pltpu.VMEM_SHARED((N, H), jnp.float32),
        pltpu.VMEM_SHARED((N_PAD, H), jnp.float32),
        [pltpu.SemaphoreType.DMA for _ in range(2)],
    ],
    compiler_params=pltpu.CompilerParams(use_tc_tiling_on_sc=False),
)


def _tch_body(x_ref, w1_ref, h_ref):
    h_ref[...] = jnp.dot(x_ref[...], w1_ref[...],
                         preferred_element_type=jnp.float32)


_tch_call = pl.pallas_call(
    _tch_body,
    out_shape=jax.ShapeDtypeStruct((N, H), jnp.float32),
)


def _tcg_body(h_ref, degp_ref, g_ref, dinv_ref):
    deg = (degp_ref[0, 0:N, 0] + degp_ref[1, 0:N, 0]).astype(jnp.float32) + 1.0
    dinv = lax.rsqrt(deg)
    g_ref[...] = h_ref[...] * dinv[:, None]
    dinv_ref[...] = dinv[:, None]


_tcg_call = pl.pallas_call(
    _tcg_body,
    out_shape=[
        jax.ShapeDtypeStruct((N, H), jnp.float32),
        jax.ShapeDtypeStruct((N, 1), jnp.float32),
    ],
)


def _tc2_body(accp_ref, g_ref, dinv_ref, b1_ref, w2_ref, b2_ref, out_ref):
    ssum = accp_ref[0, 0:N, :] + accp_ref[1, 0:N, :] + g_ref[...]
    act = jnp.maximum(dinv_ref[...] * ssum + b1_ref[...], 0.0)
    out_ref[...] = (
        jnp.dot(act, w2_ref[...], preferred_element_type=jnp.float32) + b2_ref[...]
    )


_tc2_call = pl.pallas_call(
    _tc2_body,
    out_shape=jax.ShapeDtypeStruct((N, 1), jnp.float32),
)


def kernel(x, edge_index, W1, b1, W2, b2):
    e3 = edge_index.reshape(2, NCH, C)
    zeros_deg = jnp.zeros((N_PAD, DW), jnp.int16)
    ones_c = jnp.ones((C, DW), jnp.int16)

    h = _tch_call(x, W1)
    degp = _deg_call(e3, zeros_deg, ones_c)
    g, dinv = _tcg_call(h, degp)
    accp = _scat_call(g, e3)
    return _tc2_call(accp, g, dinv, b1.reshape(1, H), W2, b2.reshape(1, 1))
